# baseline (speedup 1.0000x reference)
import functools

import jax
import jax.numpy as jnp
from jax import lax
from jax.experimental import pallas as pl
from jax.experimental.pallas import tpu as pltpu
from jax.experimental.pallas import tpu_sc as plsc

_NC = 2
_NS = 16


def _pick_bm(m):
    for bm in (512, 400, 256, 128, 80, 16, 8):
        if m % bm == 0:
            return bm
    return m


def _mm(x, wt, brow, relu=False):
    m, k = x.shape
    n = wt.shape[1]
    bm = _pick_bm(m)

    def body(x_ref, w_ref, b_ref, o_ref):
        acc = jnp.dot(x_ref[...], w_ref[...], preferred_element_type=jnp.float32)
        acc = acc + b_ref[...]
        if relu:
            acc = jnp.maximum(acc, 0.0)
        o_ref[...] = acc

    return pl.pallas_call(
        body,
        grid=(m // bm,),
        in_specs=[
            pl.BlockSpec((bm, k), lambda i: (i, 0)),
            pl.BlockSpec((k, n), lambda i: (0, 0)),
            pl.BlockSpec((1, n), lambda i: (0, 0)),
        ],
        out_specs=pl.BlockSpec((bm, n), lambda i: (i, 0)),
        out_shape=jax.ShapeDtypeStruct((m, n), jnp.float32),
    )(x, wt, brow)


def _mm_gcn(h, wt, dp0, dp1):
    m, k = h.shape
    n = wt.shape[1]
    bm = _pick_bm(m)

    def body(x_ref, w_ref, d0_ref, d1_ref, o_ref):
        dis = lax.rsqrt(1.0 + d0_ref[...][:, 0:1] + d1_ref[...][:, 0:1])
        acc = jnp.dot(x_ref[...], w_ref[...], preferred_element_type=jnp.float32)
        o_ref[...] = acc * dis

    return pl.pallas_call(
        body,
        grid=(m // bm,),
        in_specs=[
            pl.BlockSpec((bm, k), lambda i: (i, 0)),
            pl.BlockSpec((k, n), lambda i: (0, 0)),
            pl.BlockSpec((bm, 128), lambda i: (i, 0)),
            pl.BlockSpec((bm, 128), lambda i: (i, 0)),
        ],
        out_specs=pl.BlockSpec((bm, n), lambda i: (i, 0)),
        out_shape=jax.ShapeDtypeStruct((m, n), jnp.float32),
    )(h, wt, dp0, dp1)


def _gcn_epilogue(m0, m1, hs, dp0, dp1, brow):
    m, n = hs.shape
    bm = _pick_bm(m)

    def body(a_ref, b_ref, hs_ref, d0_ref, d1_ref, bias_ref, o_ref):
        dis = lax.rsqrt(1.0 + d0_ref[...][:, 0:1] + d1_ref[...][:, 0:1])
        o_ref[...] = (a_ref[...] + b_ref[...] + hs_ref[...]) * dis + bias_ref[...]

    return pl.pallas_call(
        body,
        grid=(m // bm,),
        in_specs=[
            pl.BlockSpec((bm, n), lambda i: (i, 0)),
            pl.BlockSpec((bm, n), lambda i: (i, 0)),
            pl.BlockSpec((bm, n), lambda i: (i, 0)),
            pl.BlockSpec((bm, 128), lambda i: (i, 0)),
            pl.BlockSpec((bm, 128), lambda i: (i, 0)),
            pl.BlockSpec((1, n), lambda i: (0, 0)),
        ],
        out_specs=pl.BlockSpec((bm, n), lambda i: (i, 0)),
        out_shape=jax.ShapeDtypeStruct((m, n), jnp.float32),
    )(m0, m1, hs, dp0, dp1, brow)


def _pool_seg(h, batch3d, wkq_t, bkq_row, wv_t, bv_row, qc_flat, b_sz, c_sz):
    n, d = h.shape
    nb = _pick_bm(n)
    nblk = n // nb
    cd = c_sz * d
    scale = 1.0 / (d ** 0.5)

    def body(h_ref, b3_ref, wkq_ref, bkq_ref, wv_ref, bv_ref, qc_ref,
             pool_ref, seg_ref):
        i = pl.program_id(0)

        @pl.when(i == 0)
        def _():
            pool_ref[...] = jnp.broadcast_to(qc_ref[...], (b_sz, cd))
            seg_ref[...] = jnp.zeros((b_sz, d), jnp.float32)

        hb = h_ref[...]
        sc = (jnp.dot(hb, wkq_ref[...], preferred_element_type=jnp.float32)
              + bkq_ref[...]) * scale
        mx = jnp.max(sc, axis=1, keepdims=True)
        e = jnp.exp(sc - mx)
        s = e / jnp.sum(e, axis=1, keepdims=True)
        vb = jnp.dot(hb, wv_ref[...], preferred_element_type=jnp.float32) + bv_ref[...]
        bt = b3_ref[0, 0, :]
        oh = (lax.broadcasted_iota(jnp.int32, (b_sz, nb), 0)
              == bt[None, :]).astype(jnp.float32)
        outer = (s[:, :, None] * vb[:, None, :]).reshape(nb, cd)
        pool_ref[...] += jnp.dot(oh, outer, preferred_element_type=jnp.float32)
        seg_ref[...] += jnp.dot(oh, hb, preferred_element_type=jnp.float32)

    return pl.pallas_call(
        body,
        grid=(nblk,),
        in_specs=[
            pl.BlockSpec((nb, d), lambda i: (i, 0)),
            pl.BlockSpec((1, 1, nb), lambda i: (i, 0, 0)),
            pl.BlockSpec((d, c_sz), lambda i: (0, 0)),
            pl.BlockSpec((1, c_sz), lambda i: (0, 0)),
            pl.BlockSpec((d, d), lambda i: (0, 0)),
            pl.BlockSpec((1, d), lambda i: (0, 0)),
            pl.BlockSpec((1, cd), lambda i: (0, 0)),
        ],
        out_specs=[
            pl.BlockSpec((b_sz, cd), lambda i: (0, 0)),
            pl.BlockSpec((b_sz, d), lambda i: (0, 0)),
        ],
        out_shape=[
            jax.ShapeDtypeStruct((b_sz, cd), jnp.float32),
            jax.ShapeDtypeStruct((b_sz, d), jnp.float32),
        ],
    )(h, batch3d, wkq_t, bkq_row, wv_t, bv_row, qc_flat)


def _pool_epilogue(x, wo_t, bo_row):
    m, k = x.shape
    n = wo_t.shape[1]
    bm = _pick_bm(m)

    def body(x_ref, w_ref, b_ref, o_ref):
        y = jnp.dot(x_ref[...], w_ref[...], preferred_element_type=jnp.float32)
        y = jnp.maximum(y + b_ref[...], 0.0)
        nrm = jnp.sqrt(jnp.sum(y * y, axis=1, keepdims=True))
        o_ref[...] = y / jnp.maximum(nrm, 1e-12)

    return pl.pallas_call(
        body,
        grid=(m // bm,),
        in_specs=[
            pl.BlockSpec((bm, k), lambda i: (i, 0)),
            pl.BlockSpec((k, n), lambda i: (0, 0)),
            pl.BlockSpec((1, n), lambda i: (0, 0)),
        ],
        out_specs=pl.BlockSpec((bm, n), lambda i: (i, 0)),
        out_shape=jax.ShapeDtypeStruct((m, n), jnp.float32),
    )(x, wo_t, bo_row)


def _sim(p1, p2):
    b, c, d = p1.shape
    bb = _pick_bm(b)

    def body(a_ref, b_ref, o_ref):
        o_ref[...] = lax.dot_general(
            a_ref[...], b_ref[...], (((2,), (2,)), ((0,), (0,))),
            preferred_element_type=jnp.float32)

    return pl.pallas_call(
        body,
        grid=(b // bb,),
        in_specs=[
            pl.BlockSpec((bb, c, d), lambda i: (i, 0, 0)),
            pl.BlockSpec((bb, c, d), lambda i: (i, 0, 0)),
        ],
        out_specs=pl.BlockSpec((bb, c, c), lambda i: (i, 0, 0)),
        out_shape=jax.ShapeDtypeStruct((b, c, c), jnp.float32),
    )(p1, p2)


@functools.partial(jax.jit, static_argnames=())
def _sc_hist(dst3, zer, ones_rows):
    nw, ch, km = dst3.shape
    npad, width = zer.shape
    rps = npad // _NS
    mesh = plsc.VectorSubcoreMesh(core_axis_name="c", subcore_axis_name="s",
                                  num_cores=_NC, num_subcores=_NS)

    @functools.partial(
        pl.kernel,
        out_type=jax.ShapeDtypeStruct((_NC, npad, width), jnp.float32),
        mesh=mesh,
        scratch_types=[
            pltpu.VMEM((ch, km), jnp.int32),
            pltpu.VMEM((km, width), jnp.float32),
            pltpu.VMEM_SHARED((npad, width), jnp.float32),
            pltpu.SemaphoreType.DMA,
        ],
    )
    def k(dst_hbm, zer_hbm, ones_hbm, out_hbm, idx_d, ones_v, acc, sem):
        c = lax.axis_index("c")
        s = lax.axis_index("s")
        wid = s * _NC + c
        pltpu.sync_copy(zer_hbm.at[pl.ds(s * rps, rps)], acc.at[pl.ds(s * rps, rps)])
        pltpu.sync_copy(dst_hbm.at[wid], idx_d)
        pltpu.sync_copy(ones_hbm, ones_v)
        plsc.subcore_barrier()

        def fire(j, carry):
            pltpu.async_copy(ones_v, acc.at[idx_d.at[j]], sem, add=True)
            return carry

        lax.fori_loop(0, ch, fire, 0)

        def drain(j, carry):
            pltpu.make_async_copy(ones_v, acc.at[idx_d.at[j]], sem).wait()
            return carry

        lax.fori_loop(0, ch, drain, 0)
        plsc.subcore_barrier()
        pltpu.sync_copy(acc.at[pl.ds(s * rps, rps)],
                        out_hbm.at[c, pl.ds(s * rps, rps)])

    return k(dst3, zer, ones_rows)


@jax.jit
def _sc_scatter_rows(hs, src3, dst3, zer):
    n, d = hs.shape
    npad = zer.shape[0]
    nw, ch, km = src3.shape
    rps = npad // _NS
    mesh = plsc.VectorSubcoreMesh(core_axis_name="c", subcore_axis_name="s",
                                  num_cores=_NC, num_subcores=_NS)

    @functools.partial(
        pl.kernel,
        out_type=jax.ShapeDtypeStruct((_NC, npad, d), jnp.float32),
        mesh=mesh,
        scratch_types=[
            pltpu.VMEM((ch, km), jnp.int32),
            pltpu.VMEM((ch, km), jnp.int32),
            pltpu.VMEM((km, d), jnp.float32),
            pltpu.VMEM_SHARED((npad, d), jnp.float32),
            pltpu.SemaphoreType.DMA,
        ],
    )
    def k(hs_hbm, src_hbm, dst_hbm, zer_hbm, out_hbm,
          idx_s, idx_d, rows_a, acc, sem_a):
        c = lax.axis_index("c")
        s = lax.axis_index("s")
        wid = s * _NC + c
        pltpu.sync_copy(zer_hbm.at[pl.ds(s * rps, rps)], acc.at[pl.ds(s * rps, rps)])
        pltpu.sync_copy(src_hbm.at[wid], idx_s)
        pltpu.sync_copy(dst_hbm.at[wid], idx_d)
        plsc.subcore_barrier()

        def chunk(j, carry):
            pltpu.async_copy(hs_hbm.at[idx_s.at[j]], rows_a, sem_a).wait()
            pltpu.sync_copy(rows_a, acc.at[idx_d.at[j]], add=True)
            return carry

        lax.fori_loop(0, ch, chunk, 0)

        plsc.subcore_barrier()
        pltpu.sync_copy(acc.at[pl.ds(s * rps, rps)],
                        out_hbm.at[c, pl.ds(s * rps, rps)])

    return k(hs, src3, dst3, zer)


def _encode(x, edge_index, batch, pp, zer_nd, ones_rows, b_sz):
    n = x.shape[0]
    d = pp["node_wt"].shape[1]
    c_sz = pp["c_sz"]
    e = edge_index.shape[1]
    nw, km = _NC * _NS, 128
    epad = -(-e // (nw * km)) * nw * km
    ch = epad // (nw * km)
    src3 = jnp.concatenate(
        [edge_index[0], jnp.zeros((epad - e,), edge_index.dtype)]).reshape(nw, ch, km)
    dst3 = jnp.concatenate(
        [edge_index[1], jnp.full((epad - e,), n, edge_index.dtype)]).reshape(nw, ch, km)

    degp = _sc_hist(dst3, zer_nd, ones_rows)
    dp0, dp1 = degp[0], degp[1]

    h = _mm(x, pp["node_wt"], pp["node_b"])
    for wt, brow in pp["gcn"]:
        hs = _mm_gcn(h, wt, dp0, dp1)
        msg = _sc_scatter_rows(hs, src3, dst3, zer_nd)
        h = _gcn_epilogue(msg[0], msg[1], hs, dp0, dp1, brow)

    batch3d = batch.reshape(n // _pick_bm(n), 1, _pick_bm(n))
    pool_acc, seg = _pool_seg(h, batch3d, pp["wkq_t"], pp["bkq_row"],
                              pp["wv_t"], pp["bv_row"], pp["qc_flat"],
                              b_sz, c_sz)
    pool = _pool_epilogue(pool_acc.reshape(b_sz * c_sz, d),
                          pp["wo_t"], pp["bo_row"])
    return seg, pool.reshape(b_sz, c_sz, d)


def _prep_params(params):
    p = params["pool"]
    d = params["node_W"].shape[0]
    qc = p["Q"][0] @ p["Wq"].T + p["bq"]
    c_sz = qc.shape[0]
    g1 = params["m1_g"] / jnp.sqrt(1.0 + 1e-5)
    g2 = params["m2_g"] / jnp.sqrt(1.0 + 1e-5)
    return {
        "c_sz": c_sz,
        "node_wt": params["node_W"].T,
        "node_b": params["node_b"].reshape(1, -1),
        "gcn": [(g["W"].T, g["b"].reshape(1, -1)) for g in params["gcn"]],
        "wkq_t": (qc @ p["Wk"]).T,
        "bkq_row": (qc @ p["bk"]).reshape(1, -1),
        "wv_t": p["Wv"].T,
        "bv_row": p["bv"].reshape(1, -1),
        "qc_flat": qc.reshape(1, -1),
        "wo_t": p["Wo"].T,
        "bo_row": p["bo"].reshape(1, -1),
        "m0_wt": params["m0_W"].T,
        "m0_b": params["m0_b"].reshape(1, -1),
        "m1_wt": params["m1_W"].T * g1[None, :],
        "m1_b": (params["m1_b"] * g1 + params["m1_be"]).reshape(1, -1),
        "m2_wt": params["m2_W"].T * g2[None, :],
        "m2_b": (params["m2_b"] * g2 + params["m2_be"]).reshape(1, -1),
        "m3f_wt": params["m3_W"].T @ params["mf_W"].T,
        "m3f_b": (params["m3_b"] @ params["mf_W"].T
                  + params["mf_b"]).reshape(1, -1),
    }


def kernel(x1, edge_index1, batch1, x2, edge_index2, batch2,
           x3, edge_index3, batch3, x4, edge_index4, batch4,
           ddi_type, params):
    n, fi = x1.shape
    d = params["node_W"].shape[0]
    b_sz = ddi_type.shape[0]
    t = params["m0_W"].shape[1] - 2 * d - params["pool"]["Q"].shape[1] ** 2

    pp = _prep_params(params)
    npad = -(-n // 128) * 128
    zer_nd = jnp.zeros((npad, d), jnp.float32)
    ones_rows = jnp.ones((128, 128), jnp.float32)

    o1, p1 = _encode(x1, edge_index1, batch1, pp, zer_nd, ones_rows, b_sz)
    o2, p2 = _encode(x2, edge_index2, batch2, pp, zer_nd, ones_rows, b_sz)
    o3, p3 = _encode(x3, edge_index3, batch3, pp, zer_nd, ones_rows, b_sz)
    o4, p4 = _encode(x4, edge_index4, batch4, pp, zer_nd, ones_rows, b_sz)

    s12 = _sim(p1, p2).reshape(b_sz, -1)
    s34 = _sim(p3, p4).reshape(b_sz, -1)
    oh = jax.nn.one_hot(ddi_type, t, dtype=jnp.float32)
    xa = jnp.concatenate([o1, o2, s12, oh], axis=-1)
    xb = jnp.concatenate([o3, o4, s34, oh], axis=-1)
    xx = jnp.concatenate([xa, xb], axis=0)

    h = _mm(xx, pp["m0_wt"], pp["m0_b"])
    h = _mm(h, pp["m1_wt"], pp["m1_b"], relu=True)
    h = _mm(h, pp["m2_wt"], pp["m2_b"], relu=True)
    s = _mm(h, pp["m3f_wt"], pp["m3f_b"])
    scores = s.reshape(2, b_sz)
    return jnp.mean(jax.nn.sigmoid(scores), axis=0)


# speedup vs baseline: 1.1596x; 1.1596x over previous
import functools

import jax
import jax.numpy as jnp
from jax import lax
from jax.experimental import pallas as pl
from jax.experimental.pallas import tpu as pltpu
from jax.experimental.pallas import tpu_sc as plsc

_NC = 2
_NS = 16


def _pick_bm(m):
    for bm in (512, 400, 256, 128, 80, 16, 8):
        if m % bm == 0:
            return bm
    return m


def _mm(x, wt, brow, relu=False):
    m, k = x.shape
    n = wt.shape[1]
    bm = _pick_bm(m)

    def body(x_ref, w_ref, b_ref, o_ref):
        acc = jnp.dot(x_ref[...], w_ref[...], preferred_element_type=jnp.float32)
        acc = acc + b_ref[...]
        if relu:
            acc = jnp.maximum(acc, 0.0)
        o_ref[...] = acc

    return pl.pallas_call(
        body,
        grid=(m // bm,),
        in_specs=[
            pl.BlockSpec((bm, k), lambda i: (i, 0)),
            pl.BlockSpec((k, n), lambda i: (0, 0)),
            pl.BlockSpec((1, n), lambda i: (0, 0)),
        ],
        out_specs=pl.BlockSpec((bm, n), lambda i: (i, 0)),
        out_shape=jax.ShapeDtypeStruct((m, n), jnp.float32),
    )(x, wt, brow)


def _mm_gcn(h, wt, dp0, dp1):
    m, k = h.shape
    n = wt.shape[1]
    bm = _pick_bm(m)

    def body(x_ref, w_ref, d0_ref, d1_ref, o_ref):
        dis = lax.rsqrt(1.0 + d0_ref[...][:, 0:1] + d1_ref[...][:, 0:1])
        acc = jnp.dot(x_ref[...], w_ref[...], preferred_element_type=jnp.float32)
        o_ref[...] = acc * dis

    return pl.pallas_call(
        body,
        grid=(m // bm,),
        in_specs=[
            pl.BlockSpec((bm, k), lambda i: (i, 0)),
            pl.BlockSpec((k, n), lambda i: (0, 0)),
            pl.BlockSpec((bm, 128), lambda i: (i, 0)),
            pl.BlockSpec((bm, 128), lambda i: (i, 0)),
        ],
        out_specs=pl.BlockSpec((bm, n), lambda i: (i, 0)),
        out_shape=jax.ShapeDtypeStruct((m, n), jnp.float32),
    )(h, wt, dp0, dp1)


def _gcn_epilogue(m0, m1, hs, dp0, dp1, brow):
    m, n = hs.shape
    bm = _pick_bm(m)

    def body(a_ref, b_ref, hs_ref, d0_ref, d1_ref, bias_ref, o_ref):
        dis = lax.rsqrt(1.0 + d0_ref[...][:, 0:1] + d1_ref[...][:, 0:1])
        o_ref[...] = (a_ref[...] + b_ref[...] + hs_ref[...]) * dis + bias_ref[...]

    return pl.pallas_call(
        body,
        grid=(m // bm,),
        in_specs=[
            pl.BlockSpec((bm, n), lambda i: (i, 0)),
            pl.BlockSpec((bm, n), lambda i: (i, 0)),
            pl.BlockSpec((bm, n), lambda i: (i, 0)),
            pl.BlockSpec((bm, 128), lambda i: (i, 0)),
            pl.BlockSpec((bm, 128), lambda i: (i, 0)),
            pl.BlockSpec((1, n), lambda i: (0, 0)),
        ],
        out_specs=pl.BlockSpec((bm, n), lambda i: (i, 0)),
        out_shape=jax.ShapeDtypeStruct((m, n), jnp.float32),
    )(m0, m1, hs, dp0, dp1, brow)


def _pool_seg(h, batch3d, wkq_t, bkq_row, wv_t, bv_row, qc_flat, b_sz, c_sz):
    n, d = h.shape
    nb = _pick_bm(n)
    nblk = n // nb
    cd = c_sz * d
    scale = 1.0 / (d ** 0.5)

    def body(h_ref, b3_ref, wkq_ref, bkq_ref, wv_ref, bv_ref, qc_ref,
             pool_ref, seg_ref):
        i = pl.program_id(0)

        @pl.when(i == 0)
        def _():
            pool_ref[...] = jnp.broadcast_to(qc_ref[...], (b_sz, cd))
            seg_ref[...] = jnp.zeros((b_sz, d), jnp.float32)

        hb = h_ref[...]
        sc = (jnp.dot(hb, wkq_ref[...], preferred_element_type=jnp.float32)
              + bkq_ref[...]) * scale
        mx = jnp.max(sc, axis=1, keepdims=True)
        e = jnp.exp(sc - mx)
        s = e / jnp.sum(e, axis=1, keepdims=True)
        vb = jnp.dot(hb, wv_ref[...], preferred_element_type=jnp.float32) + bv_ref[...]
        bt = b3_ref[0, 0, :]
        oh = (lax.broadcasted_iota(jnp.int32, (b_sz, nb), 0)
              == bt[None, :]).astype(jnp.float32)
        outer = (s[:, :, None] * vb[:, None, :]).reshape(nb, cd)
        pool_ref[...] += jnp.dot(oh, outer, preferred_element_type=jnp.float32)
        seg_ref[...] += jnp.dot(oh, hb, preferred_element_type=jnp.float32)

    return pl.pallas_call(
        body,
        grid=(nblk,),
        in_specs=[
            pl.BlockSpec((nb, d), lambda i: (i, 0)),
            pl.BlockSpec((1, 1, nb), lambda i: (i, 0, 0)),
            pl.BlockSpec((d, c_sz), lambda i: (0, 0)),
            pl.BlockSpec((1, c_sz), lambda i: (0, 0)),
            pl.BlockSpec((d, d), lambda i: (0, 0)),
            pl.BlockSpec((1, d), lambda i: (0, 0)),
            pl.BlockSpec((1, cd), lambda i: (0, 0)),
        ],
        out_specs=[
            pl.BlockSpec((b_sz, cd), lambda i: (0, 0)),
            pl.BlockSpec((b_sz, d), lambda i: (0, 0)),
        ],
        out_shape=[
            jax.ShapeDtypeStruct((b_sz, cd), jnp.float32),
            jax.ShapeDtypeStruct((b_sz, d), jnp.float32),
        ],
    )(h, batch3d, wkq_t, bkq_row, wv_t, bv_row, qc_flat)


def _pool_epilogue(x, wo_t, bo_row):
    m, k = x.shape
    n = wo_t.shape[1]
    bm = _pick_bm(m)

    def body(x_ref, w_ref, b_ref, o_ref):
        y = jnp.dot(x_ref[...], w_ref[...], preferred_element_type=jnp.float32)
        y = jnp.maximum(y + b_ref[...], 0.0)
        nrm = jnp.sqrt(jnp.sum(y * y, axis=1, keepdims=True))
        o_ref[...] = y / jnp.maximum(nrm, 1e-12)

    return pl.pallas_call(
        body,
        grid=(m // bm,),
        in_specs=[
            pl.BlockSpec((bm, k), lambda i: (i, 0)),
            pl.BlockSpec((k, n), lambda i: (0, 0)),
            pl.BlockSpec((1, n), lambda i: (0, 0)),
        ],
        out_specs=pl.BlockSpec((bm, n), lambda i: (i, 0)),
        out_shape=jax.ShapeDtypeStruct((m, n), jnp.float32),
    )(x, wo_t, bo_row)


def _sim(p1, p2):
    b, c, d = p1.shape
    bb = _pick_bm(b)

    def body(a_ref, b_ref, o_ref):
        o_ref[...] = lax.dot_general(
            a_ref[...], b_ref[...], (((2,), (2,)), ((0,), (0,))),
            preferred_element_type=jnp.float32)

    return pl.pallas_call(
        body,
        grid=(b // bb,),
        in_specs=[
            pl.BlockSpec((bb, c, d), lambda i: (i, 0, 0)),
            pl.BlockSpec((bb, c, d), lambda i: (i, 0, 0)),
        ],
        out_specs=pl.BlockSpec((bb, c, c), lambda i: (i, 0, 0)),
        out_shape=jax.ShapeDtypeStruct((b, c, c), jnp.float32),
    )(p1, p2)


@functools.partial(jax.jit, static_argnames=())
def _sc_hist(dst3, zer, ones_rows):
    nw, ch, km = dst3.shape
    npad, width = zer.shape
    rps = npad // _NS
    mesh = plsc.VectorSubcoreMesh(core_axis_name="c", subcore_axis_name="s",
                                  num_cores=_NC, num_subcores=_NS)

    @functools.partial(
        pl.kernel,
        out_type=jax.ShapeDtypeStruct((_NC, npad, width), jnp.float32),
        mesh=mesh,
        scratch_types=[
            pltpu.VMEM((ch, km), jnp.int32),
            pltpu.VMEM((km, width), jnp.float32),
            pltpu.VMEM_SHARED((npad, width), jnp.float32),
            pltpu.SemaphoreType.DMA,
        ],
    )
    def k(dst_hbm, zer_hbm, ones_hbm, out_hbm, idx_d, ones_v, acc, sem):
        c = lax.axis_index("c")
        s = lax.axis_index("s")
        wid = s * _NC + c
        pltpu.sync_copy(zer_hbm.at[pl.ds(s * rps, rps)], acc.at[pl.ds(s * rps, rps)])
        pltpu.sync_copy(dst_hbm.at[wid], idx_d)
        pltpu.sync_copy(ones_hbm, ones_v)
        plsc.subcore_barrier()

        def fire(j, carry):
            pltpu.async_copy(ones_v, acc.at[idx_d.at[j]], sem, add=True)
            return carry

        lax.fori_loop(0, ch, fire, 0)

        def drain(j, carry):
            pltpu.make_async_copy(ones_v, acc.at[idx_d.at[j]], sem).wait()
            return carry

        lax.fori_loop(0, ch, drain, 0)
        plsc.subcore_barrier()
        pltpu.sync_copy(acc.at[pl.ds(s * rps, rps)],
                        out_hbm.at[c, pl.ds(s * rps, rps)])

    return k(dst3, zer, ones_rows)


@jax.jit
def _sc_scatter_rows(hs, src3, dst3, zer):
    n, d = hs.shape
    npad = zer.shape[0]
    nw, ch, km = src3.shape
    rps = npad // _NS
    mesh = plsc.VectorSubcoreMesh(core_axis_name="c", subcore_axis_name="s",
                                  num_cores=_NC, num_subcores=_NS)

    @functools.partial(
        pl.kernel,
        out_type=jax.ShapeDtypeStruct((_NC, npad, d), jnp.float32),
        mesh=mesh,
        scratch_types=[
            pltpu.VMEM((km,), jnp.int32),
            pltpu.VMEM((km,), jnp.int32),
            pltpu.VMEM((ch, km), jnp.int32),
            pltpu.VMEM((km, d), jnp.float32),
            pltpu.VMEM((km, d), jnp.float32),
            pltpu.VMEM_SHARED((npad, d), jnp.float32),
            pltpu.SemaphoreType.DMA,
            pltpu.SemaphoreType.DMA,
            pltpu.SemaphoreType.DMA,
            pltpu.SemaphoreType.DMA,
        ],
    )
    def k(hs_hbm, src_hbm, dst_hbm, zer_hbm, out_hbm,
          is_a, is_b, idx_d, rows_a, rows_b, acc, sia, sib, sga, sgb):
        c = lax.axis_index("c")
        s = lax.axis_index("s")
        wid = s * _NC + c
        pltpu.sync_copy(zer_hbm.at[pl.ds(s * rps, rps)], acc.at[pl.ds(s * rps, rps)])
        pltpu.sync_copy(dst_hbm.at[wid], idx_d)
        plsc.subcore_barrier()

        pltpu.async_copy(src_hbm.at[wid, 0], is_a, sia)
        pltpu.async_copy(src_hbm.at[wid, 1], is_b, sib)
        pltpu.make_async_copy(src_hbm.at[wid, 0], is_a, sia).wait()
        pltpu.async_copy(hs_hbm.at[is_a], rows_a, sga)

        def pair(t, carry):
            j0 = 2 * t
            pltpu.make_async_copy(hs_hbm.at[is_a], rows_a, sga).wait()

            @pl.when(j0 + 2 < ch)
            def _():
                pltpu.async_copy(src_hbm.at[wid, j0 + 2], is_a, sia)

            pltpu.make_async_copy(src_hbm.at[wid, j0 + 1], is_b, sib).wait()
            pltpu.async_copy(hs_hbm.at[is_b], rows_b, sgb)
            pltpu.sync_copy(rows_a, acc.at[idx_d.at[j0]], add=True)
            pltpu.make_async_copy(hs_hbm.at[is_b], rows_b, sgb).wait()

            @pl.when(j0 + 3 < ch)
            def _():
                pltpu.async_copy(src_hbm.at[wid, j0 + 3], is_b, sib)

            @pl.when(j0 + 2 < ch)
            def _():
                pltpu.make_async_copy(src_hbm.at[wid, j0 + 2], is_a, sia).wait()
                pltpu.async_copy(hs_hbm.at[is_a], rows_a, sga)

            pltpu.sync_copy(rows_b, acc.at[idx_d.at[j0 + 1]], add=True)
            return carry

        lax.fori_loop(0, ch // 2, pair, 0)

        plsc.subcore_barrier()
        pltpu.sync_copy(acc.at[pl.ds(s * rps, rps)],
                        out_hbm.at[c, pl.ds(s * rps, rps)])

    return k(hs, src3, dst3, zer)


def _encode(x, edge_index, batch, pp, zer_nd, ones_rows, b_sz):
    n = x.shape[0]
    d = pp["node_wt"].shape[1]
    c_sz = pp["c_sz"]
    e = edge_index.shape[1]
    nw, km = _NC * _NS, 128
    epad = -(-e // (nw * km)) * nw * km
    ch = epad // (nw * km)
    src3 = jnp.concatenate(
        [edge_index[0], jnp.zeros((epad - e,), edge_index.dtype)]).reshape(nw, ch, km)
    dst3 = jnp.concatenate(
        [edge_index[1], jnp.full((epad - e,), n, edge_index.dtype)]).reshape(nw, ch, km)

    degp = _sc_hist(dst3, zer_nd, ones_rows)
    dp0, dp1 = degp[0], degp[1]

    h = _mm(x, pp["node_wt"], pp["node_b"])
    for wt, brow in pp["gcn"]:
        hs = _mm_gcn(h, wt, dp0, dp1)
        msg = _sc_scatter_rows(hs, src3, dst3, zer_nd)
        h = _gcn_epilogue(msg[0], msg[1], hs, dp0, dp1, brow)

    batch3d = batch.reshape(n // _pick_bm(n), 1, _pick_bm(n))
    pool_acc, seg = _pool_seg(h, batch3d, pp["wkq_t"], pp["bkq_row"],
                              pp["wv_t"], pp["bv_row"], pp["qc_flat"],
                              b_sz, c_sz)
    pool = _pool_epilogue(pool_acc.reshape(b_sz * c_sz, d),
                          pp["wo_t"], pp["bo_row"])
    return seg, pool.reshape(b_sz, c_sz, d)


def _prep_params(params):
    p = params["pool"]
    d = params["node_W"].shape[0]
    qc = p["Q"][0] @ p["Wq"].T + p["bq"]
    c_sz = qc.shape[0]
    g1 = params["m1_g"] / jnp.sqrt(1.0 + 1e-5)
    g2 = params["m2_g"] / jnp.sqrt(1.0 + 1e-5)
    return {
        "c_sz": c_sz,
        "node_wt": params["node_W"].T,
        "node_b": params["node_b"].reshape(1, -1),
        "gcn": [(g["W"].T, g["b"].reshape(1, -1)) for g in params["gcn"]],
        "wkq_t": (qc @ p["Wk"]).T,
        "bkq_row": (qc @ p["bk"]).reshape(1, -1),
        "wv_t": p["Wv"].T,
        "bv_row": p["bv"].reshape(1, -1),
        "qc_flat": qc.reshape(1, -1),
        "wo_t": p["Wo"].T,
        "bo_row": p["bo"].reshape(1, -1),
        "m0_wt": params["m0_W"].T,
        "m0_b": params["m0_b"].reshape(1, -1),
        "m1_wt": params["m1_W"].T * g1[None, :],
        "m1_b": (params["m1_b"] * g1 + params["m1_be"]).reshape(1, -1),
        "m2_wt": params["m2_W"].T * g2[None, :],
        "m2_b": (params["m2_b"] * g2 + params["m2_be"]).reshape(1, -1),
        "m3f_wt": params["m3_W"].T @ params["mf_W"].T,
        "m3f_b": (params["m3_b"] @ params["mf_W"].T
                  + params["mf_b"]).reshape(1, -1),
    }


def kernel(x1, edge_index1, batch1, x2, edge_index2, batch2,
           x3, edge_index3, batch3, x4, edge_index4, batch4,
           ddi_type, params):
    n, fi = x1.shape
    d = params["node_W"].shape[0]
    b_sz = ddi_type.shape[0]
    t = params["m0_W"].shape[1] - 2 * d - params["pool"]["Q"].shape[1] ** 2

    pp = _prep_params(params)
    npad = -(-n // 128) * 128
    zer_nd = jnp.zeros((npad, d), jnp.float32)
    ones_rows = jnp.ones((128, 128), jnp.float32)

    o1, p1 = _encode(x1, edge_index1, batch1, pp, zer_nd, ones_rows, b_sz)
    o2, p2 = _encode(x2, edge_index2, batch2, pp, zer_nd, ones_rows, b_sz)
    o3, p3 = _encode(x3, edge_index3, batch3, pp, zer_nd, ones_rows, b_sz)
    o4, p4 = _encode(x4, edge_index4, batch4, pp, zer_nd, ones_rows, b_sz)

    s12 = _sim(p1, p2).reshape(b_sz, -1)
    s34 = _sim(p3, p4).reshape(b_sz, -1)
    oh = jax.nn.one_hot(ddi_type, t, dtype=jnp.float32)
    xa = jnp.concatenate([o1, o2, s12, oh], axis=-1)
    xb = jnp.concatenate([o3, o4, s34, oh], axis=-1)
    xx = jnp.concatenate([xa, xb], axis=0)

    h = _mm(xx, pp["m0_wt"], pp["m0_b"])
    h = _mm(h, pp["m1_wt"], pp["m1_b"], relu=True)
    h = _mm(h, pp["m2_wt"], pp["m2_b"], relu=True)
    s = _mm(h, pp["m3f_wt"], pp["m3f_b"])
    scores = s.reshape(2, b_sz)
    return jnp.mean(jax.nn.sigmoid(scores), axis=0)
